# baseline (speedup 1.0000x reference)
import functools

import jax
import jax.numpy as jnp
from jax import lax
from jax.experimental import pallas as pl
from jax.experimental.pallas import tpu as pltpu
from jax.experimental.pallas import tpu_sc as plsc

_CAP = 65536
_D = 512
_NW = 32
_TAIL = 16384
_S = _CAP - _TAIL
_RPW = _TAIL // _NW
_CH = 64
_NCH = _RPW // _CH
_MPW = _CAP // _NW
_BLK = 8192
_GRID = _S // _BLK
_BIG_I = 2 ** 30
_BIG_F = 3.0e38
_L = 16


def _sc_body(mask_hbm, wsflat_hbm, outf_hbm, outi_hbm, outt_hbm,
             mstat, mtail, ew, buf0, buf1, stage_f, stage_i, acc_st,
             sem0, sem1):
    wid = lax.axis_index("s") * 2 + lax.axis_index("c")
    lanes = lax.iota(jnp.int32, _L)

    base = wid * _MPW
    pltpu.sync_copy(mask_hbm.at[pl.ds(base, _MPW)], mstat)

    def stat_step(k, carry):
        vmin, vidx, cidx, cval, sesum = carry
        m = mstat[pl.ds(k * _L, _L)]
        jv = lanes + (base + k * _L)
        lt = m < vmin
        eq = m == vmin
        vidx = jnp.where(lt, jv, jnp.where(eq, jnp.minimum(vidx, jv), vidx))
        vmin = jnp.minimum(vmin, m)
        take = (m < 0.5) & (cidx == _BIG_I)
        cidx = jnp.where(take, jv, cidx)
        cval = jnp.where(take, m, cval)
        sesum = sesum + jnp.exp(m)
        return vmin, vidx, cidx, cval, sesum

    init = (jnp.full((_L,), _BIG_F, jnp.float32),
            jnp.full((_L,), _BIG_I, jnp.int32),
            jnp.full((_L,), _BIG_I, jnp.int32),
            jnp.full((_L,), _BIG_F, jnp.float32),
            jnp.zeros((_L,), jnp.float32))
    vmin, vidx, cidx, cval, sesum = lax.fori_loop(
        0, _MPW // _L, stat_step, init)

    stage_f[pl.ds(0, _L)] = vmin
    stage_f[pl.ds(_L, _L)] = cval
    stage_f[pl.ds(2 * _L, _L)] = sesum
    stage_i[pl.ds(0, _L)] = vidx
    stage_i[pl.ds(_L, _L)] = cidx
    pltpu.sync_copy(stage_f, outf_hbm.at[wid])
    pltpu.sync_copy(stage_i, outi_hbm.at[wid])

    row0 = _S + wid * _RPW
    pltpu.sync_copy(mask_hbm.at[pl.ds(row0, _RPW)], mtail)

    def ew_step(k, _):
        ew[pl.ds(k * _L, _L)] = jnp.exp(mtail[pl.ds(k * _L, _L)])
        return 0
    lax.fori_loop(0, _RPW // _L, ew_step, 0)

    bufs = (buf0, buf1)
    sems = (sem0, sem1)

    def chunk_src(c):
        return wsflat_hbm.at[pl.ds((row0 + c * _CH) * _D, _CH * _D)]

    pltpu.async_copy(chunk_src(0), bufs[0], sems[0])
    pltpu.async_copy(chunk_src(1), bufs[1], sems[1])

    accs = tuple(jnp.zeros((_L,), jnp.float32) for _ in range(_D // _L))
    for c in range(_NCH):
        buf = bufs[c % 2]
        pltpu.make_async_copy(chunk_src(c), buf, sems[c % 2]).wait()

        def row_step(r, acc):
            w = ew[pl.ds(c * _CH + r, _L)][0]
            rb = r * _D
            return tuple(acc[k] + w * buf[pl.ds(rb + k * _L, _L)]
                         for k in range(_D // _L))
        accs = lax.fori_loop(0, _CH, row_step, accs)

        if c + 2 < _NCH:
            pltpu.async_copy(chunk_src(c + 2), bufs[c % 2], sems[c % 2])

    for k in range(_D // _L):
        acc_st[pl.ds(k * _L, _L)] = accs[k]
    pltpu.sync_copy(acc_st, outt_hbm.at[wid])


def _sc_call(workspace_mask, ws_flat):
    mesh = plsc.VectorSubcoreMesh(core_axis_name="c", subcore_axis_name="s")
    fn = functools.partial(
        pl.kernel, mesh=mesh,
        out_type=[
            jax.ShapeDtypeStruct((_NW, 3 * _L), jnp.float32),
            jax.ShapeDtypeStruct((_NW, 2 * _L), jnp.int32),
            jax.ShapeDtypeStruct((_NW, _D), jnp.float32),
        ],
        scratch_types=[
            pltpu.VMEM((_MPW,), jnp.float32),
            pltpu.VMEM((_RPW,), jnp.float32),
            pltpu.VMEM((_RPW + _L,), jnp.float32),
            pltpu.VMEM((_CH * _D,), jnp.float32),
            pltpu.VMEM((_CH * _D,), jnp.float32),
            pltpu.VMEM((3 * _L,), jnp.float32),
            pltpu.VMEM((2 * _L,), jnp.int32),
            pltpu.VMEM((_D,), jnp.float32),
            pltpu.SemaphoreType.DMA,
            pltpu.SemaphoreType.DMA,
        ],
    )(_sc_body)
    return fn(workspace_mask, ws_flat)


def _tc_body(wrow_ref, ws_ref, acc_out_ref, acc_ref):
    i = pl.program_id(0)

    @pl.when(i == 0)
    def _init():
        acc_ref[...] = jnp.zeros_like(acc_ref)

    w = jnp.exp(wrow_ref[...])
    acc_ref[...] += lax.dot_general(
        w, ws_ref[...], (((1,), (0,)), ((), ())),
        preferred_element_type=jnp.float32,
        precision=lax.Precision.DEFAULT)

    @pl.when(i == _GRID - 1)
    def _final():
        acc_out_ref[...] = acc_ref[...]


def _tc_call(workspace, wrow):
    return pl.pallas_call(
        _tc_body,
        grid=(_GRID,),
        in_specs=[
            pl.BlockSpec((1, _BLK), lambda i: (0, i)),
            pl.BlockSpec((_BLK, _D), lambda i: (i, 0)),
        ],
        out_specs=pl.BlockSpec((1, _D), lambda i: (0, 0)),
        out_shape=jax.ShapeDtypeStruct((1, _D), jnp.float32),
        scratch_shapes=[pltpu.VMEM((1, _D), jnp.float32)],
    )(wrow, workspace)


def _comb_body(acc_ref, statsf_ref, statsi_ref, tails_ref, sal_ref,
               content_ref, W_ref, b_ref, ws_any, out_ref, row_ref, sem):
    gm = statsf_ref[:, 0:_L]
    cval = statsf_ref[:, _L:2 * _L]
    se = statsf_ref[:, 2 * _L:3 * _L]
    gidx = statsi_ref[:, 0:_L]
    cmin = statsi_ref[:, _L:2 * _L]

    GM = jnp.min(gm, keepdims=True)
    GIDX = jnp.min(jnp.where(gm == GM, gidx, _BIG_I))
    CMIN = jnp.min(cmin)
    has_avail = CMIN < _BIG_I
    CMIN11 = jnp.min(cmin, keepdims=True)
    CVAL = jnp.min(jnp.where(cmin == CMIN11, cval, _BIG_F), keepdims=True)
    SE = jnp.sum(se, keepdims=True)

    idx = jnp.where(has_avail, CMIN, GIDX)
    m_idx = jnp.where(has_avail, CVAL, GM)

    copy = pltpu.make_async_copy(ws_any.at[pl.ds(idx, 1), :], row_ref, sem)
    copy.start()
    copy.wait()

    emi = jnp.exp(m_idx)
    es = jnp.exp(sal_ref[...])
    tail_sum = jnp.sum(tails_ref[...], axis=0, keepdims=True)
    num = (acc_ref[...] + tail_sum - emi * row_ref[...]
           + es * content_ref[...])
    den = SE - emi + es
    out_ref[...] = lax.dot_general(
        num / den, W_ref[...], (((1,), (1,)), ((), ())),
        preferred_element_type=jnp.float32,
        precision=lax.Precision.HIGHEST) + b_ref[...]


def _comb_call(acc, statsf, statsi, tails, sal, content, W, b, workspace):
    return pl.pallas_call(
        _comb_body,
        in_specs=[
            pl.BlockSpec((1, _D), lambda: (0, 0)),
            pl.BlockSpec((_NW, 3 * _L), lambda: (0, 0)),
            pl.BlockSpec((_NW, 2 * _L), lambda: (0, 0)),
            pl.BlockSpec((_NW, _D), lambda: (0, 0)),
            pl.BlockSpec((1, 1), lambda: (0, 0)),
            pl.BlockSpec((1, _D), lambda: (0, 0)),
            pl.BlockSpec((_D, _D), lambda: (0, 0)),
            pl.BlockSpec((1, _D), lambda: (0, 0)),
            pl.BlockSpec(memory_space=pl.ANY),
        ],
        out_specs=pl.BlockSpec((1, _D), lambda: (0, 0)),
        out_shape=jax.ShapeDtypeStruct((1, _D), jnp.float32),
        scratch_shapes=[
            pltpu.VMEM((1, _D), jnp.float32),
            pltpu.SemaphoreType.DMA,
        ],
    )(acc, statsf, statsi, tails, sal, content, W, b, workspace)


@jax.jit
def kernel(content, salience, workspace, workspace_mask, W, b):
    ws_flat = workspace.reshape(-1)
    wrow = workspace_mask.reshape(1, _CAP)
    sal = salience.reshape(1, 1)
    cont = content.reshape(1, _D)
    bb = b.reshape(1, _D)

    statsf, statsi, tails = _sc_call(workspace_mask, ws_flat)
    acc = _tc_call(workspace, wrow)
    out = _comb_call(acc, statsf, statsi, tails, sal, cont, W, bb, workspace)
    return out.reshape(_D)


# speedup vs baseline: 2.5407x; 2.5407x over previous
import functools

import jax
import jax.numpy as jnp
from jax import lax
from jax.experimental import pallas as pl
from jax.experimental.pallas import tpu as pltpu
from jax.experimental.pallas import tpu_sc as plsc

_CAP = 65536
_D = 512
_NW = 32
_TAIL = 16384
_S = _CAP - _TAIL
_RPW = _TAIL // _NW
_CH = 64
_NCH = _RPW // _CH
_MPW = _CAP // _NW
_BLK = 8192
_GRID = _S // _BLK
_BIG_I = 2 ** 30
_BIG_F = 3.0e38
_L = 16


def _sc_body(mask_hbm, wsflat_hbm, outf_hbm, outi_hbm, outt_hbm,
             mstat, mtail, ew, buf0, buf1, stage_f, stage_i, acc_st,
             sem0, sem1):
    wid = lax.axis_index("s") * 2 + lax.axis_index("c")
    lanes = lax.iota(jnp.int32, _L)

    base = wid * _MPW
    pltpu.sync_copy(mask_hbm.at[pl.ds(base, _MPW)], mstat)

    def stat_step(k, carry):
        vmin, vidx, cidx, cval, sesum = carry
        m = mstat[pl.ds(k * _L, _L)]
        jv = lanes + (base + k * _L)
        lt = m < vmin
        eq = m == vmin
        vidx = jnp.where(lt, jv, jnp.where(eq, jnp.minimum(vidx, jv), vidx))
        vmin = jnp.minimum(vmin, m)
        take = (m < 0.5) & (cidx == _BIG_I)
        cidx = jnp.where(take, jv, cidx)
        cval = jnp.where(take, m, cval)
        sesum = sesum + jnp.exp(m)
        return vmin, vidx, cidx, cval, sesum

    init = (jnp.full((_L,), _BIG_F, jnp.float32),
            jnp.full((_L,), _BIG_I, jnp.int32),
            jnp.full((_L,), _BIG_I, jnp.int32),
            jnp.full((_L,), _BIG_F, jnp.float32),
            jnp.zeros((_L,), jnp.float32))
    vmin, vidx, cidx, cval, sesum = lax.fori_loop(
        0, _MPW // _L, stat_step, init)

    stage_f[pl.ds(0, _L)] = vmin
    stage_f[pl.ds(_L, _L)] = cval
    stage_f[pl.ds(2 * _L, _L)] = sesum
    stage_i[pl.ds(0, _L)] = vidx
    stage_i[pl.ds(_L, _L)] = cidx
    pltpu.sync_copy(stage_f, outf_hbm.at[wid])
    pltpu.sync_copy(stage_i, outi_hbm.at[wid])

    row0 = _S + wid * _RPW
    pltpu.sync_copy(mask_hbm.at[pl.ds(row0, _RPW)], mtail)

    def ew_step(k, _):
        ew[pl.ds(k * _L, _L)] = jnp.exp(mtail[pl.ds(k * _L, _L)])
        return 0
    lax.fori_loop(0, _RPW // _L, ew_step, 0)

    bufs = (buf0, buf1)
    sems = (sem0, sem1)

    def chunk_src(c):
        return wsflat_hbm.at[pl.ds(row0 + c * _CH, _CH), :]

    pltpu.async_copy(chunk_src(0), bufs[0], sems[0])
    pltpu.async_copy(chunk_src(1), bufs[1], sems[1])

    accs = tuple(jnp.zeros((_L,), jnp.float32) for _ in range(_D // _L))
    for c in range(_NCH):
        buf = bufs[c % 2]
        pltpu.make_async_copy(chunk_src(c), buf, sems[c % 2]).wait()

        def row_step(r, acc):
            w = ew[pl.ds(c * _CH + r, _L)][0]
            return tuple(acc[k] + w * buf[r, pl.ds(k * _L, _L)]
                         for k in range(_D // _L))
        accs = lax.fori_loop(0, _CH, row_step, accs)

        if c + 2 < _NCH:
            pltpu.async_copy(chunk_src(c + 2), bufs[c % 2], sems[c % 2])

    for k in range(_D // _L):
        acc_st[pl.ds(k * _L, _L)] = accs[k]
    pltpu.sync_copy(acc_st, outt_hbm.at[wid])


def _sc_call(workspace_mask, ws_flat):
    mesh = plsc.VectorSubcoreMesh(core_axis_name="c", subcore_axis_name="s")
    fn = functools.partial(
        pl.kernel, mesh=mesh,
        out_type=[
            jax.ShapeDtypeStruct((_NW, 3 * _L), jnp.float32),
            jax.ShapeDtypeStruct((_NW, 2 * _L), jnp.int32),
            jax.ShapeDtypeStruct((_NW, _D), jnp.float32),
        ],
        scratch_types=[
            pltpu.VMEM((_MPW,), jnp.float32),
            pltpu.VMEM((_RPW,), jnp.float32),
            pltpu.VMEM((_RPW + _L,), jnp.float32),
            pltpu.VMEM((_CH, _D), jnp.float32),
            pltpu.VMEM((_CH, _D), jnp.float32),
            pltpu.VMEM((3 * _L,), jnp.float32),
            pltpu.VMEM((2 * _L,), jnp.int32),
            pltpu.VMEM((_D,), jnp.float32),
            pltpu.SemaphoreType.DMA,
            pltpu.SemaphoreType.DMA,
        ],
    )(_sc_body)
    return fn(workspace_mask, ws_flat)


def _tc_body(wrow_ref, ws_ref, acc_out_ref, acc_ref):
    i = pl.program_id(0)

    @pl.when(i == 0)
    def _init():
        acc_ref[...] = jnp.zeros_like(acc_ref)

    w = jnp.exp(wrow_ref[...])
    acc_ref[...] += lax.dot_general(
        w, ws_ref[...], (((1,), (0,)), ((), ())),
        preferred_element_type=jnp.float32,
        precision=lax.Precision.DEFAULT)

    @pl.when(i == _GRID - 1)
    def _final():
        acc_out_ref[...] = acc_ref[...]


def _tc_call(workspace, wrow):
    return pl.pallas_call(
        _tc_body,
        grid=(_GRID,),
        in_specs=[
            pl.BlockSpec((1, _BLK), lambda i: (0, i)),
            pl.BlockSpec((_BLK, _D), lambda i: (i, 0)),
        ],
        out_specs=pl.BlockSpec((1, _D), lambda i: (0, 0)),
        out_shape=jax.ShapeDtypeStruct((1, _D), jnp.float32),
        scratch_shapes=[pltpu.VMEM((1, _D), jnp.float32)],
    )(wrow, workspace)


def _comb_body(acc_ref, statsf_ref, statsi_ref, tails_ref, sal_ref,
               content_ref, W_ref, b_ref, ws_any, out_ref, row_ref, sem):
    gm = statsf_ref[:, 0:_L]
    cval = statsf_ref[:, _L:2 * _L]
    se = statsf_ref[:, 2 * _L:3 * _L]
    gidx = statsi_ref[:, 0:_L]
    cmin = statsi_ref[:, _L:2 * _L]

    GM = jnp.min(gm, keepdims=True)
    GIDX = jnp.min(jnp.where(gm == GM, gidx, _BIG_I))
    CMIN = jnp.min(cmin)
    has_avail = CMIN < _BIG_I
    CMIN11 = jnp.min(cmin, keepdims=True)
    CVAL = jnp.min(jnp.where(cmin == CMIN11, cval, _BIG_F), keepdims=True)
    SE = jnp.sum(se, keepdims=True)

    idx = jnp.where(has_avail, CMIN, GIDX)
    m_idx = jnp.where(has_avail, CVAL, GM)

    copy = pltpu.make_async_copy(ws_any.at[pl.ds(idx, 1), :], row_ref, sem)
    copy.start()
    copy.wait()

    emi = jnp.exp(m_idx)
    es = jnp.exp(sal_ref[...])
    tail_sum = jnp.sum(tails_ref[...], axis=0, keepdims=True)
    num = (acc_ref[...] + tail_sum - emi * row_ref[...]
           + es * content_ref[...])
    den = SE - emi + es
    out_ref[...] = lax.dot_general(
        num / den, W_ref[...], (((1,), (1,)), ((), ())),
        preferred_element_type=jnp.float32,
        precision=lax.Precision.HIGHEST) + b_ref[...]


def _comb_call(acc, statsf, statsi, tails, sal, content, W, b, workspace):
    return pl.pallas_call(
        _comb_body,
        in_specs=[
            pl.BlockSpec((1, _D), lambda: (0, 0)),
            pl.BlockSpec((_NW, 3 * _L), lambda: (0, 0)),
            pl.BlockSpec((_NW, 2 * _L), lambda: (0, 0)),
            pl.BlockSpec((_NW, _D), lambda: (0, 0)),
            pl.BlockSpec((1, 1), lambda: (0, 0)),
            pl.BlockSpec((1, _D), lambda: (0, 0)),
            pl.BlockSpec((_D, _D), lambda: (0, 0)),
            pl.BlockSpec((1, _D), lambda: (0, 0)),
            pl.BlockSpec(memory_space=pl.ANY),
        ],
        out_specs=pl.BlockSpec((1, _D), lambda: (0, 0)),
        out_shape=jax.ShapeDtypeStruct((1, _D), jnp.float32),
        scratch_shapes=[
            pltpu.VMEM((1, _D), jnp.float32),
            pltpu.SemaphoreType.DMA,
        ],
    )(acc, statsf, statsi, tails, sal, content, W, b, workspace)


@jax.jit
def kernel(content, salience, workspace, workspace_mask, W, b):
    wrow = workspace_mask.reshape(1, _CAP)
    sal = salience.reshape(1, 1)
    cont = content.reshape(1, _D)
    bb = b.reshape(1, _D)

    statsf, statsi, tails = _sc_call(workspace_mask, workspace)
    acc = _tc_call(workspace, wrow)
    out = _comb_call(acc, statsf, statsi, tails, sal, cont, W, bb, workspace)
    return out.reshape(_D)


# speedup vs baseline: 2.6012x; 1.0238x over previous
import functools

import jax
import jax.numpy as jnp
from jax import lax
from jax.experimental import pallas as pl
from jax.experimental.pallas import tpu as pltpu
from jax.experimental.pallas import tpu_sc as plsc

_CAP = 65536
_D = 512
_NW = 32
_TAIL = 4096
_S = _CAP - _TAIL
_RPW = _TAIL // _NW
_CH = 64
_NCH = _RPW // _CH
_MPW = _CAP // _NW
_BLK = 8192
_GRID = _S // _BLK
_BIG_I = 2 ** 30
_BIG_F = 3.0e38
_L = 16


def _sc_body(mask_hbm, wsflat_hbm, outf_hbm, outi_hbm, outt_hbm,
             mstat, mtail, ew, buf0, buf1, stage_f, stage_i, acc_st,
             sem0, sem1):
    wid = lax.axis_index("s") * 2 + lax.axis_index("c")
    lanes = lax.iota(jnp.int32, _L)

    base = wid * _MPW
    pltpu.sync_copy(mask_hbm.at[pl.ds(base, _MPW)], mstat)

    def stat_step(k, carry):
        vmin, vidx, cidx, cval, sesum = carry
        m = mstat[pl.ds(k * _L, _L)]
        jv = lanes + (base + k * _L)
        lt = m < vmin
        eq = m == vmin
        vidx = jnp.where(lt, jv, jnp.where(eq, jnp.minimum(vidx, jv), vidx))
        vmin = jnp.minimum(vmin, m)
        take = (m < 0.5) & (cidx == _BIG_I)
        cidx = jnp.where(take, jv, cidx)
        cval = jnp.where(take, m, cval)
        sesum = sesum + jnp.exp(m)
        return vmin, vidx, cidx, cval, sesum

    init = (jnp.full((_L,), _BIG_F, jnp.float32),
            jnp.full((_L,), _BIG_I, jnp.int32),
            jnp.full((_L,), _BIG_I, jnp.int32),
            jnp.full((_L,), _BIG_F, jnp.float32),
            jnp.zeros((_L,), jnp.float32))
    vmin, vidx, cidx, cval, sesum = lax.fori_loop(
        0, _MPW // _L, stat_step, init)

    stage_f[pl.ds(0, _L)] = vmin
    stage_f[pl.ds(_L, _L)] = cval
    stage_f[pl.ds(2 * _L, _L)] = sesum
    stage_i[pl.ds(0, _L)] = vidx
    stage_i[pl.ds(_L, _L)] = cidx
    pltpu.sync_copy(stage_f, outf_hbm.at[wid])
    pltpu.sync_copy(stage_i, outi_hbm.at[wid])

    row0 = _S + wid * _RPW
    pltpu.sync_copy(mask_hbm.at[pl.ds(row0, _RPW)], mtail)

    def ew_step(k, _):
        ew[pl.ds(k * _L, _L)] = jnp.exp(mtail[pl.ds(k * _L, _L)])
        return 0
    lax.fori_loop(0, _RPW // _L, ew_step, 0)

    bufs = (buf0, buf1)
    sems = (sem0, sem1)

    def chunk_src(c):
        return wsflat_hbm.at[pl.ds(row0 + c * _CH, _CH), :]

    pltpu.async_copy(chunk_src(0), bufs[0], sems[0])
    pltpu.async_copy(chunk_src(1), bufs[1], sems[1])

    accs = tuple(jnp.zeros((_L,), jnp.float32) for _ in range(_D // _L))
    for c in range(_NCH):
        buf = bufs[c % 2]
        pltpu.make_async_copy(chunk_src(c), buf, sems[c % 2]).wait()

        def row_step(r, acc):
            w = ew[pl.ds(c * _CH + r, _L)][0]
            return tuple(acc[k] + w * buf[r, pl.ds(k * _L, _L)]
                         for k in range(_D // _L))
        accs = lax.fori_loop(0, _CH, row_step, accs)

        if c + 2 < _NCH:
            pltpu.async_copy(chunk_src(c + 2), bufs[c % 2], sems[c % 2])

    for k in range(_D // _L):
        acc_st[pl.ds(k * _L, _L)] = accs[k]
    pltpu.sync_copy(acc_st, outt_hbm.at[wid])


def _sc_call(workspace_mask, ws_flat):
    mesh = plsc.VectorSubcoreMesh(core_axis_name="c", subcore_axis_name="s")
    fn = functools.partial(
        pl.kernel, mesh=mesh,
        out_type=[
            jax.ShapeDtypeStruct((_NW, 3 * _L), jnp.float32),
            jax.ShapeDtypeStruct((_NW, 2 * _L), jnp.int32),
            jax.ShapeDtypeStruct((_NW, _D), jnp.float32),
        ],
        scratch_types=[
            pltpu.VMEM((_MPW,), jnp.float32),
            pltpu.VMEM((_RPW,), jnp.float32),
            pltpu.VMEM((_RPW + _L,), jnp.float32),
            pltpu.VMEM((_CH, _D), jnp.float32),
            pltpu.VMEM((_CH, _D), jnp.float32),
            pltpu.VMEM((3 * _L,), jnp.float32),
            pltpu.VMEM((2 * _L,), jnp.int32),
            pltpu.VMEM((_D,), jnp.float32),
            pltpu.SemaphoreType.DMA,
            pltpu.SemaphoreType.DMA,
        ],
    )(_sc_body)
    return fn(workspace_mask, ws_flat)


def _tc_body(wrow_ref, ws_ref, acc_out_ref, acc_ref):
    i = pl.program_id(0)

    @pl.when(i == 0)
    def _init():
        acc_ref[...] = jnp.zeros_like(acc_ref)

    w = jnp.exp(wrow_ref[...])
    acc_ref[...] += lax.dot_general(
        w, ws_ref[...], (((1,), (0,)), ((), ())),
        preferred_element_type=jnp.float32,
        precision=lax.Precision.DEFAULT)

    @pl.when(i == _GRID - 1)
    def _final():
        acc_out_ref[...] = acc_ref[...]


def _tc_call(workspace, wrow):
    return pl.pallas_call(
        _tc_body,
        grid=(_GRID,),
        in_specs=[
            pl.BlockSpec((1, _BLK), lambda i: (0, i)),
            pl.BlockSpec((_BLK, _D), lambda i: (i, 0)),
        ],
        out_specs=pl.BlockSpec((1, _D), lambda i: (0, 0)),
        out_shape=jax.ShapeDtypeStruct((1, _D), jnp.float32),
        scratch_shapes=[pltpu.VMEM((1, _D), jnp.float32)],
    )(wrow, workspace)


def _comb_body(acc_ref, statsf_ref, statsi_ref, tails_ref, sal_ref,
               content_ref, W_ref, b_ref, ws_any, out_ref, row_ref, sem):
    gm = statsf_ref[:, 0:_L]
    cval = statsf_ref[:, _L:2 * _L]
    se = statsf_ref[:, 2 * _L:3 * _L]
    gidx = statsi_ref[:, 0:_L]
    cmin = statsi_ref[:, _L:2 * _L]

    GM = jnp.min(gm, keepdims=True)
    GIDX = jnp.min(jnp.where(gm == GM, gidx, _BIG_I))
    CMIN = jnp.min(cmin)
    has_avail = CMIN < _BIG_I
    CMIN11 = jnp.min(cmin, keepdims=True)
    CVAL = jnp.min(jnp.where(cmin == CMIN11, cval, _BIG_F), keepdims=True)
    SE = jnp.sum(se, keepdims=True)

    idx = jnp.where(has_avail, CMIN, GIDX)
    m_idx = jnp.where(has_avail, CVAL, GM)

    copy = pltpu.make_async_copy(ws_any.at[pl.ds(idx, 1), :], row_ref, sem)
    copy.start()
    copy.wait()

    emi = jnp.exp(m_idx)
    es = jnp.exp(sal_ref[...])
    tail_sum = jnp.sum(tails_ref[...], axis=0, keepdims=True)
    num = (acc_ref[...] + tail_sum - emi * row_ref[...]
           + es * content_ref[...])
    den = SE - emi + es
    out_ref[...] = lax.dot_general(
        num / den, W_ref[...], (((1,), (1,)), ((), ())),
        preferred_element_type=jnp.float32,
        precision=lax.Precision.HIGHEST) + b_ref[...]


def _comb_call(acc, statsf, statsi, tails, sal, content, W, b, workspace):
    return pl.pallas_call(
        _comb_body,
        in_specs=[
            pl.BlockSpec((1, _D), lambda: (0, 0)),
            pl.BlockSpec((_NW, 3 * _L), lambda: (0, 0)),
            pl.BlockSpec((_NW, 2 * _L), lambda: (0, 0)),
            pl.BlockSpec((_NW, _D), lambda: (0, 0)),
            pl.BlockSpec((1, 1), lambda: (0, 0)),
            pl.BlockSpec((1, _D), lambda: (0, 0)),
            pl.BlockSpec((_D, _D), lambda: (0, 0)),
            pl.BlockSpec((1, _D), lambda: (0, 0)),
            pl.BlockSpec(memory_space=pl.ANY),
        ],
        out_specs=pl.BlockSpec((1, _D), lambda: (0, 0)),
        out_shape=jax.ShapeDtypeStruct((1, _D), jnp.float32),
        scratch_shapes=[
            pltpu.VMEM((1, _D), jnp.float32),
            pltpu.SemaphoreType.DMA,
        ],
    )(acc, statsf, statsi, tails, sal, content, W, b, workspace)


@jax.jit
def kernel(content, salience, workspace, workspace_mask, W, b):
    wrow = workspace_mask.reshape(1, _CAP)
    sal = salience.reshape(1, 1)
    cont = content.reshape(1, _D)
    bb = b.reshape(1, _D)

    statsf, statsi, tails = _sc_call(workspace_mask, workspace)
    acc = _tc_call(workspace, wrow)
    out = _comb_call(acc, statsf, statsi, tails, sal, cont, W, bb, workspace)
    return out.reshape(_D)
